# baseline (speedup 1.0000x reference)
import jax
import jax.numpy as jnp
from jax import lax
from jax.experimental import pallas as pl
from jax.experimental.pallas import tpu as pltpu
from jax.experimental.pallas import tpu_sc as plsc

N = 10000
E = 160000
F = 256
P = 4
FC = 128
NCHUNK = (F * P) // FC
NSC = 2
NSUB = 16
NW = NSC * NSUB
EPW = E // NW
B = 128
NB = -(-EPW // B)
EPW_PAD = NB * B
SROWS = 10240
TRASH = N
ZROWS = SROWS // NSUB
ZB = 32
BN = 1000
GRID = N // BN

_MESH = dict(core_axis_name="c", subcore_axis_name="s", num_cores=NSC,
             num_subcores=NSUB)


def _sc_deg_body(cols_hbm, ones_hbm, zeros_hbm, out_hbm,
                 cols_v, ones_v, zb_v, deg_sp):
  c = lax.axis_index("c")
  s = lax.axis_index("s")
  wid = c * NSUB + s
  pltpu.sync_copy(zeros_hbm, zb_v)
  pltpu.sync_copy(ones_hbm, ones_v)
  pltpu.sync_copy(cols_hbm.at[wid], cols_v)

  def zero(i, carry):
    pltpu.sync_copy(zb_v, deg_sp.at[pl.ds(s * ZROWS + i * ZB, ZB)])
    return carry

  lax.fori_loop(0, ZROWS // ZB, zero, 0)
  plsc.subcore_barrier()

  def batch(b, carry):
    pltpu.sync_copy(ones_v, deg_sp.at[cols_v.at[b]], add=True)
    return carry

  lax.fori_loop(0, NB, batch, 0)
  plsc.subcore_barrier()
  pltpu.sync_copy(deg_sp.at[pl.ds(s * ZROWS, ZROWS)],
                  out_hbm.at[c, pl.ds(s * ZROWS, ZROWS)])


def _sc_deg(cols_p, ones128, zeros128):
  k = pl.kernel(
      _sc_deg_body,
      out_type=jax.ShapeDtypeStruct((NSC, SROWS, FC), jnp.float32),
      mesh=plsc.VectorSubcoreMesh(**_MESH),
      scratch_types=[
          pltpu.VMEM((NB, B), jnp.int32),
          pltpu.VMEM((B, FC), jnp.float32),
          pltpu.VMEM((ZB, FC), jnp.float32),
          pltpu.VMEM_SHARED((SROWS, FC), jnp.float32),
      ],
  )
  return k(cols_p, ones128, zeros128)


def _sc_spmm_body(xs_hbm, rows8_hbm, cols_hbm, zeros_hbm, out_hbm,
                  rowsc_v, cols_v, g0_v, g1_v, zb_v, s_sp, sem0, sem1):
  c = lax.axis_index("c")
  s = lax.axis_index("s")
  wid = c * NSUB + s
  pltpu.sync_copy(zeros_hbm, zb_v)
  pltpu.sync_copy(cols_hbm.at[wid], cols_v)

  def gather(b, buf, sem):
    pltpu.async_copy(xs_hbm.at[rowsc_v.at[b]], buf, sem)

  def gwait(buf, sem):
    pltpu.make_async_copy(xs_hbm.at[rowsc_v.at[0]], buf, sem).wait()

  def zero(i, carry):
    pltpu.sync_copy(zb_v, s_sp.at[pl.ds(s * ZROWS + i * ZB, ZB)])
    return carry

  pltpu.sync_copy(rows8_hbm.at[0, wid], rowsc_v)
  gather(0, g0_v, sem0)
  gather(1, g1_v, sem1)
  lax.fori_loop(0, ZROWS // ZB, zero, 0)
  plsc.subcore_barrier()

  for chunk in range(NCHUNK):
    def pair(i, carry):
      b0 = 2 * i
      gwait(g0_v, sem0)
      pltpu.sync_copy(g0_v, s_sp.at[cols_v.at[b0]], add=True)

      @pl.when(b0 + 2 < NB)
      def _():
        gather(b0 + 2, g0_v, sem0)

      gwait(g1_v, sem1)
      pltpu.sync_copy(g1_v, s_sp.at[cols_v.at[b0 + 1]], add=True)

      @pl.when(b0 + 3 < NB)
      def _():
        gather(b0 + 3, g1_v, sem1)

      return carry

    lax.fori_loop(0, NB // 2, pair, 0)

    if chunk + 1 < NCHUNK:
      pltpu.sync_copy(rows8_hbm.at[chunk + 1, wid], rowsc_v)
      gather(0, g0_v, sem0)
      gather(1, g1_v, sem1)

    plsc.subcore_barrier()
    pltpu.sync_copy(s_sp.at[pl.ds(s * ZROWS, ZROWS)],
                    out_hbm.at[c, chunk, pl.ds(s * ZROWS, ZROWS)])
    if chunk + 1 < NCHUNK:
      lax.fori_loop(0, ZROWS // ZB, zero, 0)
    plsc.subcore_barrier()


def _sc_spmm(xs_flat, rows8_p, cols_p, zeros128):
  k = pl.kernel(
      _sc_spmm_body,
      out_type=jax.ShapeDtypeStruct((NSC, NCHUNK, SROWS, FC), jnp.float32),
      mesh=plsc.VectorSubcoreMesh(**_MESH),
      scratch_types=[
          pltpu.VMEM((NB, B), jnp.int32),
          pltpu.VMEM((NB, B), jnp.int32),
          pltpu.VMEM((B, FC), jnp.float32),
          pltpu.VMEM((B, FC), jnp.float32),
          pltpu.VMEM((ZB, FC), jnp.float32),
          pltpu.VMEM_SHARED((SROWS, FC), jnp.float32),
          pltpu.SemaphoreType.DMA,
          pltpu.SemaphoreType.DMA,
      ],
  )
  return k(xs_flat, rows8_p, cols_p, zeros128)


def _tc_scale_body(xt_ref, degp_ref, xs_ref, dinv_ref):
  d = degp_ref[0, :, 0:1] + degp_ref[1, :, 0:1] + 1.0
  dinv = lax.rsqrt(d)
  dinv_ref[...] = dinv
  xs_ref[...] = xt_ref[0] * dinv


def _tc_scale(xt8, degp):
  return pl.pallas_call(
      _tc_scale_body,
      grid=(GRID, NCHUNK),
      in_specs=[
          pl.BlockSpec((1, BN, FC), lambda i, c: (c, i, 0)),
          pl.BlockSpec((NSC, BN, FC), lambda i, c: (0, i, 0)),
      ],
      out_specs=[
          pl.BlockSpec((BN, FC), lambda i, c: (c * (N // BN) + i, 0)),
          pl.BlockSpec((BN, 1), lambda i, c: (i, 0)),
      ],
      out_shape=[
          jax.ShapeDtypeStruct((NCHUNK * N, FC), jnp.float32),
          jax.ShapeDtypeStruct((N, 1), jnp.float32),
      ],
  )(xt8, degp)


def _tc_final_body(sp_ref, xt_ref, dinv_ref, att_ref, wz_ref, bz_ref,
                   lzw_ref, lzb_ref, wh_ref, bh_ref, lhw_ref, lhb_ref,
                   out_ref, mz_s, mh_s, cz_s, ch_s):
  prec = lax.Precision.HIGHEST

  @pl.when(pl.program_id(0) == 0)
  def _():
    mz_s[...] = jnp.dot(wz_ref[...], lzw_ref[0:F, :],
                        preferred_element_type=jnp.float32, precision=prec)
    mh_s[...] = jnp.dot(wh_ref[...], lhw_ref[0:F, :],
                        preferred_element_type=jnp.float32, precision=prec)
    cz_s[...] = jnp.dot(bz_ref[...], lzw_ref[0:F, :],
                        preferred_element_type=jnp.float32,
                        precision=prec) + lzb_ref[...]
    ch_s[...] = jnp.dot(bh_ref[...], lhw_ref[0:F, :],
                        preferred_element_type=jnp.float32,
                        precision=prec) + lhb_ref[...]

  att = att_ref[...]
  e = jnp.exp(att - jnp.max(att, axis=1, keepdims=True))
  probs = e / jnp.sum(e, axis=1, keepdims=True)
  dinv = dinv_ref[...]
  s8 = sp_ref[0] + sp_ref[1] + xt_ref[...] * dinv[None, :, :]

  prec_big = lax.Precision.HIGHEST
  acc = jnp.zeros((BN, F), jnp.float32)
  for p in range(P):
    wp = 0.5 * jnp.sum(probs[:, p:p + 1])
    yp = jnp.concatenate([s8[2 * p], s8[2 * p + 1]], axis=-1) * dinv
    az = jnp.dot(yp, mz_s[...], preferred_element_type=jnp.float32,
                 precision=prec_big) + cz_s[...]
    ah = jnp.dot(yp, mh_s[...], preferred_element_type=jnp.float32,
                 precision=prec_big) + ch_s[...]
    sig = 1.0 / (1.0 + jnp.exp(-az))
    acc = acc + wp * ((1.0 - sig) * jnp.tanh(ah))
  out_ref[...] = acc


def _tc_final(sp, xt8, dinv, attention, Wz, bz, lzW, lzb, Wh, bh, lhW, lhb):
  def full(shape):
    return pl.BlockSpec(shape, lambda i, _s=shape: tuple(0 for _ in _s))

  return pl.pallas_call(
      _tc_final_body,
      grid=(GRID,),
      in_specs=[
          pl.BlockSpec((NSC, NCHUNK, BN, FC), lambda i: (0, 0, i, 0)),
          pl.BlockSpec((NCHUNK, BN, FC), lambda i: (0, i, 0)),
          pl.BlockSpec((BN, 1), lambda i: (i, 0)),
          full((2, P)),
          full((F, F)),
          full((1, F)),
          full((2 * F, F)),
          full((1, F)),
          full((F, F)),
          full((1, F)),
          full((2 * F, F)),
          full((1, F)),
      ],
      out_specs=pl.BlockSpec((BN, F), lambda i: (i, 0)),
      out_shape=jax.ShapeDtypeStruct((N, F), jnp.float32),
      scratch_shapes=[
          pltpu.VMEM((F, F), jnp.float32),
          pltpu.VMEM((F, F), jnp.float32),
          pltpu.VMEM((1, F), jnp.float32),
          pltpu.VMEM((1, F), jnp.float32),
      ],
  )(sp, xt8, dinv, attention, Wz, bz, lzW, lzb, Wh, bh, lhW, lhb)


def kernel(X, edge_index, attention, Wz, bz, lzW, lzb, Wr, br, lrW, lrb,
           Wh, bh, lhW, lhb):
  del Wr, br, lrW, lrb
  rows = edge_index[0].reshape(NW, EPW)
  cols = edge_index[1].reshape(NW, EPW)
  pad = EPW_PAD - EPW
  pad_rows = ((jnp.arange(NW, dtype=jnp.int32)[:, None] * 997 +
               jnp.arange(pad, dtype=jnp.int32)[None, :] * 131) % N)
  pad_cols = (TRASH +
              (jnp.arange(NW, dtype=jnp.int32)[:, None] * 7 +
               jnp.arange(pad, dtype=jnp.int32)[None, :]) % (SROWS - N))
  rows_p = jnp.concatenate([rows, pad_rows], axis=1).reshape(NW, NB, B)
  cols_p = jnp.concatenate([cols, pad_cols], axis=1).reshape(NW, NB, B)
  rows8_p = (rows_p[None] +
             (jnp.arange(NCHUNK, dtype=jnp.int32) * N)[:, None, None, None])

  xt8 = (X.transpose(2, 0, 1)
         .reshape(P, N, 2, FC)
         .transpose(0, 2, 1, 3)
         .reshape(NCHUNK, N, FC))

  ones128 = jnp.ones((B, FC), jnp.float32)
  zeros128 = jnp.zeros((ZB, FC), jnp.float32)

  degp = _sc_deg(cols_p, ones128, zeros128)
  xs_flat, dinv = _tc_scale(xt8, degp)
  sp = _sc_spmm(xs_flat, rows8_p, cols_p, zeros128)
  return _tc_final(sp, xt8, dinv, attention, Wz, bz.reshape(1, F), lzW,
                   lzb.reshape(1, F), Wh, bh.reshape(1, F), lhW,
                   lhb.reshape(1, F))


# speedup vs baseline: 1.0992x; 1.0992x over previous
import jax
import jax.numpy as jnp
from jax import lax
from jax.experimental import pallas as pl
from jax.experimental.pallas import tpu as pltpu
from jax.experimental.pallas import tpu_sc as plsc

N = 10000
E = 160000
F = 256
P = 4
FC = 128
NCHUNK = (F * P) // FC
NSC = 2
NSUB = 16
NW = NSC * NSUB
EPW = E // NW
B = 128
NB = -(-EPW // B)
EPW_PAD = NB * B
SROWS = 10240
TRASH = N
ZROWS = SROWS // NSUB
ZB = 32
BN = 1000
GRID = N // BN

_MESH = dict(core_axis_name="c", subcore_axis_name="s", num_cores=NSC,
             num_subcores=NSUB)


def _sc_deg_body(cols_hbm, ones_hbm, zeros_hbm, out_hbm,
                 cols_v, ones_v, zb_v, deg_sp):
  c = lax.axis_index("c")
  s = lax.axis_index("s")
  wid = c * NSUB + s
  pltpu.sync_copy(zeros_hbm, zb_v)
  pltpu.sync_copy(ones_hbm, ones_v)
  pltpu.sync_copy(cols_hbm.at[wid], cols_v)

  def zero(i, carry):
    pltpu.sync_copy(zb_v, deg_sp.at[pl.ds(s * ZROWS + i * ZB, ZB)])
    return carry

  lax.fori_loop(0, ZROWS // ZB, zero, 0)
  plsc.subcore_barrier()

  def batch(b, carry):
    pltpu.sync_copy(ones_v, deg_sp.at[cols_v.at[b]], add=True)
    return carry

  lax.fori_loop(0, NB, batch, 0)
  plsc.subcore_barrier()
  pltpu.sync_copy(deg_sp.at[pl.ds(s * ZROWS, ZROWS)],
                  out_hbm.at[c, pl.ds(s * ZROWS, ZROWS)])


def _sc_deg(cols_p, ones16, zeros16):
  k = pl.kernel(
      _sc_deg_body,
      out_type=jax.ShapeDtypeStruct((NSC, SROWS, 16), jnp.float32),
      mesh=plsc.VectorSubcoreMesh(**_MESH),
      compiler_params=pltpu.CompilerParams(use_tc_tiling_on_sc=False),
      scratch_types=[
          pltpu.VMEM((NB, B), jnp.int32),
          pltpu.VMEM((B, 16), jnp.float32),
          pltpu.VMEM((ZB, 16), jnp.float32),
          pltpu.VMEM_SHARED((SROWS, 16), jnp.float32),
      ],
  )
  return k(cols_p, ones16, zeros16)


def _sc_spmm_body(xs_hbm, rows8_hbm, cols_hbm, zeros_hbm, out_hbm,
                  rowsc_v, cols_v, g0_v, g1_v, zb_v, s_sp, sem0, sem1):
  c = lax.axis_index("c")
  s = lax.axis_index("s")
  wid = c * NSUB + s
  pltpu.sync_copy(zeros_hbm, zb_v)
  pltpu.sync_copy(cols_hbm.at[wid], cols_v)

  def gather(b, buf, sem):
    pltpu.async_copy(xs_hbm.at[rowsc_v.at[b]], buf, sem)

  def gwait(buf, sem):
    pltpu.make_async_copy(xs_hbm.at[rowsc_v.at[0]], buf, sem).wait()

  def zero(i, carry):
    pltpu.sync_copy(zb_v, s_sp.at[pl.ds(s * ZROWS + i * ZB, ZB)])
    return carry

  pltpu.sync_copy(rows8_hbm.at[0, wid], rowsc_v)
  gather(0, g0_v, sem0)
  gather(1, g1_v, sem1)
  lax.fori_loop(0, ZROWS // ZB, zero, 0)
  plsc.subcore_barrier()

  for chunk in range(NCHUNK):
    def pair(i, carry):
      b0 = 2 * i
      gwait(g0_v, sem0)
      pltpu.sync_copy(g0_v, s_sp.at[cols_v.at[b0]], add=True)

      @pl.when(b0 + 2 < NB)
      def _():
        gather(b0 + 2, g0_v, sem0)

      gwait(g1_v, sem1)
      pltpu.sync_copy(g1_v, s_sp.at[cols_v.at[b0 + 1]], add=True)

      @pl.when(b0 + 3 < NB)
      def _():
        gather(b0 + 3, g1_v, sem1)

      return carry

    lax.fori_loop(0, NB // 2, pair, 0)

    if chunk + 1 < NCHUNK:
      pltpu.sync_copy(rows8_hbm.at[chunk + 1, wid], rowsc_v)
      gather(0, g0_v, sem0)
      gather(1, g1_v, sem1)

    plsc.subcore_barrier()
    pltpu.sync_copy(s_sp.at[pl.ds(s * ZROWS, ZROWS)],
                    out_hbm.at[c, chunk, pl.ds(s * ZROWS, ZROWS)])
    if chunk + 1 < NCHUNK:
      lax.fori_loop(0, ZROWS // ZB, zero, 0)
    plsc.subcore_barrier()


def _sc_spmm(xs_flat, rows8_p, cols_p, zeros128):
  k = pl.kernel(
      _sc_spmm_body,
      out_type=jax.ShapeDtypeStruct((NSC, NCHUNK, SROWS, FC), jnp.float32),
      mesh=plsc.VectorSubcoreMesh(**_MESH),
      scratch_types=[
          pltpu.VMEM((NB, B), jnp.int32),
          pltpu.VMEM((NB, B), jnp.int32),
          pltpu.VMEM((B, FC), jnp.float32),
          pltpu.VMEM((B, FC), jnp.float32),
          pltpu.VMEM((ZB, FC), jnp.float32),
          pltpu.VMEM_SHARED((SROWS, FC), jnp.float32),
          pltpu.SemaphoreType.DMA,
          pltpu.SemaphoreType.DMA,
      ],
  )
  return k(xs_flat, rows8_p, cols_p, zeros128)


def _tc_scale_body(xt_ref, degp_ref, xs_ref, dinv_ref):
  d = degp_ref[0, :, 0:1] + degp_ref[1, :, 0:1] + 1.0
  dinv = lax.rsqrt(d)
  dinv_ref[...] = dinv
  xs_ref[...] = xt_ref[...] * dinv[None, :, :]


def _tc_scale(xt8, degp):
  return pl.pallas_call(
      _tc_scale_body,
      grid=(GRID,),
      in_specs=[
          pl.BlockSpec((NCHUNK, BN, FC), lambda i: (0, i, 0)),
          pl.BlockSpec((NSC, BN, 16), lambda i: (0, i, 0)),
      ],
      out_specs=[
          pl.BlockSpec((NCHUNK, BN, FC), lambda i: (0, i, 0)),
          pl.BlockSpec((BN, 1), lambda i: (i, 0)),
      ],
      out_shape=[
          jax.ShapeDtypeStruct((NCHUNK, N, FC), jnp.float32),
          jax.ShapeDtypeStruct((N, 1), jnp.float32),
      ],
  )(xt8, degp)


def _tc_final_body(sp_ref, xt_ref, dinv_ref, att_ref, wz_ref, bz_ref,
                   lzw_ref, lzb_ref, wh_ref, bh_ref, lhw_ref, lhb_ref,
                   out_ref, mz_s, mh_s, cz_s, ch_s):
  prec = lax.Precision.HIGHEST

  @pl.when(pl.program_id(0) == 0)
  def _():
    mz_s[...] = jnp.dot(wz_ref[...], lzw_ref[0:F, :],
                        preferred_element_type=jnp.float32, precision=prec)
    mh_s[...] = jnp.dot(wh_ref[...], lhw_ref[0:F, :],
                        preferred_element_type=jnp.float32, precision=prec)
    cz_s[...] = jnp.dot(bz_ref[...], lzw_ref[0:F, :],
                        preferred_element_type=jnp.float32,
                        precision=prec) + lzb_ref[...]
    ch_s[...] = jnp.dot(bh_ref[...], lhw_ref[0:F, :],
                        preferred_element_type=jnp.float32,
                        precision=prec) + lhb_ref[...]

  att = att_ref[...]
  e = jnp.exp(att - jnp.max(att, axis=1, keepdims=True))
  probs = e / jnp.sum(e, axis=1, keepdims=True)
  dinv = dinv_ref[...]
  s8 = sp_ref[0] + sp_ref[1] + xt_ref[...] * dinv[None, :, :]

  prec_big = lax.Precision.HIGHEST
  acc = jnp.zeros((BN, F), jnp.float32)
  for p in range(P):
    wp = 0.5 * jnp.sum(probs[:, p:p + 1])
    yp = jnp.concatenate([s8[2 * p], s8[2 * p + 1]], axis=-1) * dinv
    az = jnp.dot(yp, mz_s[...], preferred_element_type=jnp.float32,
                 precision=prec_big) + cz_s[...]
    ah = jnp.dot(yp, mh_s[...], preferred_element_type=jnp.float32,
                 precision=prec_big) + ch_s[...]
    sig = 1.0 / (1.0 + jnp.exp(-az))
    acc = acc + wp * ((1.0 - sig) * jnp.tanh(ah))
  out_ref[...] = acc


def _tc_final(sp, xt8, dinv, attention, Wz, bz, lzW, lzb, Wh, bh, lhW, lhb):
  def full(shape):
    return pl.BlockSpec(shape, lambda i, _s=shape: tuple(0 for _ in _s))

  return pl.pallas_call(
      _tc_final_body,
      grid=(GRID,),
      in_specs=[
          pl.BlockSpec((NSC, NCHUNK, BN, FC), lambda i: (0, 0, i, 0)),
          pl.BlockSpec((NCHUNK, BN, FC), lambda i: (0, i, 0)),
          pl.BlockSpec((BN, 1), lambda i: (i, 0)),
          full((2, P)),
          full((F, F)),
          full((1, F)),
          full((2 * F, F)),
          full((1, F)),
          full((F, F)),
          full((1, F)),
          full((2 * F, F)),
          full((1, F)),
      ],
      out_specs=pl.BlockSpec((BN, F), lambda i: (i, 0)),
      out_shape=jax.ShapeDtypeStruct((N, F), jnp.float32),
      scratch_shapes=[
          pltpu.VMEM((F, F), jnp.float32),
          pltpu.VMEM((F, F), jnp.float32),
          pltpu.VMEM((1, F), jnp.float32),
          pltpu.VMEM((1, F), jnp.float32),
      ],
  )(sp, xt8, dinv, attention, Wz, bz, lzW, lzb, Wh, bh, lhW, lhb)


def kernel(X, edge_index, attention, Wz, bz, lzW, lzb, Wr, br, lrW, lrb,
           Wh, bh, lhW, lhb):
  del Wr, br, lrW, lrb
  rows = edge_index[0].reshape(NW, EPW)
  cols = edge_index[1].reshape(NW, EPW)
  pad = EPW_PAD - EPW
  pad_rows = ((jnp.arange(NW, dtype=jnp.int32)[:, None] * 997 +
               jnp.arange(pad, dtype=jnp.int32)[None, :] * 131) % N)
  pad_cols = (TRASH +
              (jnp.arange(NW, dtype=jnp.int32)[:, None] * 7 +
               jnp.arange(pad, dtype=jnp.int32)[None, :]) % (SROWS - N))
  rows_p = jnp.concatenate([rows, pad_rows], axis=1).reshape(NW, NB, B)
  cols_p = jnp.concatenate([cols, pad_cols], axis=1).reshape(NW, NB, B)
  rows8_p = (rows_p[None] +
             (jnp.arange(NCHUNK, dtype=jnp.int32) * N)[:, None, None, None])

  xt8 = (X.transpose(2, 0, 1)
         .reshape(P, N, 2, FC)
         .transpose(0, 2, 1, 3)
         .reshape(NCHUNK, N, FC))

  ones16 = jnp.ones((B, 16), jnp.float32)
  zeros16 = jnp.zeros((ZB, 16), jnp.float32)
  zeros128 = jnp.zeros((ZB, FC), jnp.float32)

  degp = _sc_deg(cols_p, ones16, zeros16)
  xs, dinv = _tc_scale(xt8, degp)
  sp = _sc_spmm(xs.reshape(NCHUNK * N, FC), rows8_p, cols_p, zeros128)
  return _tc_final(sp, xt8, dinv, attention, Wz, bz.reshape(1, F), lzW,
                   lzb.reshape(1, F), Wh, bh.reshape(1, F), lhW,
                   lhb.reshape(1, F))


# speedup vs baseline: 1.1564x; 1.0521x over previous
import jax
import jax.numpy as jnp
from jax import lax
from jax.experimental import pallas as pl
from jax.experimental.pallas import tpu as pltpu
from jax.experimental.pallas import tpu_sc as plsc

N = 10000
E = 160000
F = 256
P = 4
FC = 128
NCHUNK = (F * P) // FC
NSC = 2
NSUB = 16
NW = NSC * NSUB
EPW = E // NW
B = 128
NB = -(-EPW // B)
EPW_PAD = NB * B
SROWS = 10240
TRASH = N
ZROWS = SROWS // NSUB
ZB = 32
BN = 1000
GRID = N // BN

_MESH = dict(core_axis_name="c", subcore_axis_name="s", num_cores=NSC,
             num_subcores=NSUB)


def _sc_deg_body(cols_hbm, ones_hbm, zeros_hbm, out_hbm,
                 cols_v, ones_v, zb_v, deg_sp):
  c = lax.axis_index("c")
  s = lax.axis_index("s")
  wid = c * NSUB + s
  pltpu.sync_copy(zeros_hbm, zb_v)
  pltpu.sync_copy(ones_hbm, ones_v)
  pltpu.sync_copy(cols_hbm.at[wid], cols_v)

  def zero(i, carry):
    pltpu.sync_copy(zb_v, deg_sp.at[pl.ds(s * ZROWS + i * ZB, ZB)])
    return carry

  lax.fori_loop(0, ZROWS // ZB, zero, 0)
  plsc.subcore_barrier()

  def batch(b, carry):
    pltpu.sync_copy(ones_v, deg_sp.at[cols_v.at[b]], add=True)
    return carry

  lax.fori_loop(0, NB, batch, 0)
  plsc.subcore_barrier()
  pltpu.sync_copy(deg_sp.at[pl.ds(s * ZROWS, ZROWS)],
                  out_hbm.at[c, pl.ds(s * ZROWS, ZROWS)])


def _sc_deg(cols_p, ones16, zeros16):
  k = pl.kernel(
      _sc_deg_body,
      out_type=jax.ShapeDtypeStruct((NSC, SROWS, 16), jnp.float32),
      mesh=plsc.VectorSubcoreMesh(**_MESH),
      compiler_params=pltpu.CompilerParams(use_tc_tiling_on_sc=False),
      scratch_types=[
          pltpu.VMEM((NB, B), jnp.int32),
          pltpu.VMEM((B, 16), jnp.float32),
          pltpu.VMEM((ZB, 16), jnp.float32),
          pltpu.VMEM_SHARED((SROWS, 16), jnp.float32),
      ],
  )
  return k(cols_p, ones16, zeros16)


def _sc_spmm_body(xs_hbm, rows8_hbm, cols_hbm, zeros_hbm, out_hbm,
                  rowsc_v, cols_v, g0_v, g1_v, zb_v, s_sp, sem0, sem1):
  c = lax.axis_index("c")
  s = lax.axis_index("s")
  wid = c * NSUB + s
  pltpu.sync_copy(zeros_hbm, zb_v)
  pltpu.sync_copy(cols_hbm.at[wid], cols_v)

  def gather(b, buf, sem):
    pltpu.async_copy(xs_hbm.at[rowsc_v.at[b]], buf, sem)

  def gwait(buf, sem):
    pltpu.make_async_copy(xs_hbm.at[rowsc_v.at[0]], buf, sem).wait()

  def zero(i, carry):
    pltpu.sync_copy(zb_v, s_sp.at[pl.ds(s * ZROWS + i * ZB, ZB)])
    return carry

  pltpu.sync_copy(rows8_hbm.at[0, wid], rowsc_v)
  gather(0, g0_v, sem0)
  gather(1, g1_v, sem1)
  lax.fori_loop(0, ZROWS // ZB, zero, 0)
  plsc.subcore_barrier()

  for chunk in range(NCHUNK):
    def pair(i, carry):
      b0 = 2 * i
      gwait(g0_v, sem0)
      pltpu.sync_copy(g0_v, s_sp.at[cols_v.at[b0]], add=True)

      @pl.when(b0 + 2 < NB)
      def _():
        gather(b0 + 2, g0_v, sem0)

      gwait(g1_v, sem1)
      pltpu.sync_copy(g1_v, s_sp.at[cols_v.at[b0 + 1]], add=True)

      @pl.when(b0 + 3 < NB)
      def _():
        gather(b0 + 3, g1_v, sem1)

      return carry

    lax.fori_loop(0, NB // 2, pair, 0)

    if chunk + 1 < NCHUNK:
      pltpu.sync_copy(rows8_hbm.at[chunk + 1, wid], rowsc_v)
      gather(0, g0_v, sem0)
      gather(1, g1_v, sem1)

    plsc.subcore_barrier()
    pltpu.sync_copy(s_sp.at[pl.ds(s * ZROWS, ZROWS)],
                    out_hbm.at[c, chunk, pl.ds(s * ZROWS, ZROWS)])
    if chunk + 1 < NCHUNK:
      lax.fori_loop(0, ZROWS // ZB, zero, 0)
    plsc.subcore_barrier()


def _sc_spmm(xs_flat, rows8_p, cols_p, zeros128):
  k = pl.kernel(
      _sc_spmm_body,
      out_type=jax.ShapeDtypeStruct((NSC, NCHUNK, SROWS, FC), jnp.float32),
      mesh=plsc.VectorSubcoreMesh(**_MESH),
      scratch_types=[
          pltpu.VMEM((NB, B), jnp.int32),
          pltpu.VMEM((NB, B), jnp.int32),
          pltpu.VMEM((B, FC), jnp.float32),
          pltpu.VMEM((B, FC), jnp.float32),
          pltpu.VMEM((ZB, FC), jnp.float32),
          pltpu.VMEM_SHARED((SROWS, FC), jnp.float32),
          pltpu.SemaphoreType.DMA,
          pltpu.SemaphoreType.DMA,
      ],
  )
  return k(xs_flat, rows8_p, cols_p, zeros128)


def _tc_scale_body(xt_ref, degp_ref, xs_ref, dinv_ref):
  d = degp_ref[0, :, 0:1] + degp_ref[1, :, 0:1] + 1.0
  dinv = lax.rsqrt(d)
  dinv_ref[...] = dinv
  xs_ref[...] = xt_ref[...] * dinv[None, :, :]


def _tc_scale(xt8, degp):
  return pl.pallas_call(
      _tc_scale_body,
      grid=(GRID,),
      in_specs=[
          pl.BlockSpec((NCHUNK, BN, FC), lambda i: (0, i, 0)),
          pl.BlockSpec((NSC, BN, 16), lambda i: (0, i, 0)),
      ],
      out_specs=[
          pl.BlockSpec((NCHUNK, BN, FC), lambda i: (0, i, 0)),
          pl.BlockSpec((BN, 1), lambda i: (i, 0)),
      ],
      out_shape=[
          jax.ShapeDtypeStruct((NCHUNK, N, FC), jnp.float32),
          jax.ShapeDtypeStruct((N, 1), jnp.float32),
      ],
  )(xt8, degp)


def _tc_final_body(sp_ref, xt_ref, dinv_ref, att_ref, wz_ref, bz_ref,
                   lzw_ref, lzb_ref, wh_ref, bh_ref, lhw_ref, lhb_ref,
                   out_ref, mz_s, mh_s, cz_s, ch_s):
  prec = lax.Precision.HIGHEST

  @pl.when(pl.program_id(0) == 0)
  def _():
    mz_s[...] = jnp.dot(wz_ref[...], lzw_ref[0:F, :],
                        preferred_element_type=jnp.float32, precision=prec)
    mh_s[...] = jnp.dot(wh_ref[...], lhw_ref[0:F, :],
                        preferred_element_type=jnp.float32, precision=prec)
    cz_s[...] = jnp.dot(bz_ref[...], lzw_ref[0:F, :],
                        preferred_element_type=jnp.float32,
                        precision=prec) + lzb_ref[...]
    ch_s[...] = jnp.dot(bh_ref[...], lhw_ref[0:F, :],
                        preferred_element_type=jnp.float32,
                        precision=prec) + lhb_ref[...]

  att = att_ref[...]
  e = jnp.exp(att - jnp.max(att, axis=1, keepdims=True))
  probs = e / jnp.sum(e, axis=1, keepdims=True)
  dinv = dinv_ref[...]
  s8 = sp_ref[0] + sp_ref[1] + xt_ref[...] * dinv[None, :, :]

  prec_big = lax.Precision.DEFAULT
  acc = jnp.zeros((BN, F), jnp.float32)
  for p in range(P):
    wp = 0.5 * jnp.sum(probs[:, p:p + 1])
    yp = jnp.concatenate([s8[2 * p], s8[2 * p + 1]], axis=-1) * dinv
    az = jnp.dot(yp, mz_s[...], preferred_element_type=jnp.float32,
                 precision=prec_big) + cz_s[...]
    ah = jnp.dot(yp, mh_s[...], preferred_element_type=jnp.float32,
                 precision=prec_big) + ch_s[...]
    sig = 1.0 / (1.0 + jnp.exp(-az))
    acc = acc + wp * ((1.0 - sig) * jnp.tanh(ah))
  out_ref[...] = acc


def _tc_final(sp, xt8, dinv, attention, Wz, bz, lzW, lzb, Wh, bh, lhW, lhb):
  def full(shape):
    return pl.BlockSpec(shape, lambda i, _s=shape: tuple(0 for _ in _s))

  return pl.pallas_call(
      _tc_final_body,
      grid=(GRID,),
      in_specs=[
          pl.BlockSpec((NSC, NCHUNK, BN, FC), lambda i: (0, 0, i, 0)),
          pl.BlockSpec((NCHUNK, BN, FC), lambda i: (0, i, 0)),
          pl.BlockSpec((BN, 1), lambda i: (i, 0)),
          full((2, P)),
          full((F, F)),
          full((1, F)),
          full((2 * F, F)),
          full((1, F)),
          full((F, F)),
          full((1, F)),
          full((2 * F, F)),
          full((1, F)),
      ],
      out_specs=pl.BlockSpec((BN, F), lambda i: (i, 0)),
      out_shape=jax.ShapeDtypeStruct((N, F), jnp.float32),
      scratch_shapes=[
          pltpu.VMEM((F, F), jnp.float32),
          pltpu.VMEM((F, F), jnp.float32),
          pltpu.VMEM((1, F), jnp.float32),
          pltpu.VMEM((1, F), jnp.float32),
      ],
  )(sp, xt8, dinv, attention, Wz, bz, lzW, lzb, Wh, bh, lhW, lhb)


def kernel(X, edge_index, attention, Wz, bz, lzW, lzb, Wr, br, lrW, lrb,
           Wh, bh, lhW, lhb):
  del Wr, br, lrW, lrb
  rows = edge_index[0].reshape(NW, EPW)
  cols = edge_index[1].reshape(NW, EPW)
  pad = EPW_PAD - EPW
  pad_rows = ((jnp.arange(NW, dtype=jnp.int32)[:, None] * 997 +
               jnp.arange(pad, dtype=jnp.int32)[None, :] * 131) % N)
  pad_cols = (TRASH +
              (jnp.arange(NW, dtype=jnp.int32)[:, None] * 7 +
               jnp.arange(pad, dtype=jnp.int32)[None, :]) % (SROWS - N))
  rows_p = jnp.concatenate([rows, pad_rows], axis=1).reshape(NW, NB, B)
  cols_p = jnp.concatenate([cols, pad_cols], axis=1).reshape(NW, NB, B)
  rows8_p = (rows_p[None] +
             (jnp.arange(NCHUNK, dtype=jnp.int32) * N)[:, None, None, None])

  xt8 = (X.transpose(2, 0, 1)
         .reshape(P, N, 2, FC)
         .transpose(0, 2, 1, 3)
         .reshape(NCHUNK, N, FC))

  ones16 = jnp.ones((B, 16), jnp.float32)
  zeros16 = jnp.zeros((ZB, 16), jnp.float32)
  zeros128 = jnp.zeros((ZB, FC), jnp.float32)

  degp = _sc_deg(cols_p, ones16, zeros16)
  xs, dinv = _tc_scale(xt8, degp)
  sp = _sc_spmm(xs.reshape(NCHUNK * N, FC), rows8_p, cols_p, zeros128)
  return _tc_final(sp, xt8, dinv, attention, Wz, bz.reshape(1, F), lzW,
                   lzb.reshape(1, F), Wh, bh.reshape(1, F), lhW,
                   lhb.reshape(1, F))
